# baseline (speedup 1.0000x reference)
import functools

import jax
import jax.numpy as jnp
from jax import lax
from jax.experimental import pallas as pl
from jax.experimental.pallas import tpu as pltpu
from jax.experimental.pallas import tpu_sc as plsc

_NC = 2
_NS = 16
_NW = _NC * _NS
_CHUNK = 16384


def _make_sc_add(total, pos_total):
    per_w = total // _NW
    n_chunks = per_w // _CHUNK
    mesh = plsc.VectorSubcoreMesh(core_axis_name="c", subcore_axis_name="s")

    @functools.partial(
        pl.kernel,
        mesh=mesh,
        out_type=jax.ShapeDtypeStruct((total,), jnp.float32),
        scratch_types=[
            pltpu.VMEM((2, _CHUNK), jnp.float32),
            pltpu.VMEM((2, _CHUNK), jnp.float32),
            pltpu.SemaphoreType.DMA((2,)),
            pltpu.SemaphoreType.DMA((2,)),
            pltpu.SemaphoreType.DMA((2,)),
        ],
    )
    def sc_add(x_hbm, pos_hbm, out_hbm, xbuf, pbuf, sem_x, sem_p, sem_o):
        wid = lax.axis_index("s") * _NC + lax.axis_index("c")
        base = wid * per_w

        def start_in(j, b):
            e = base + j * _CHUNK
            pe = lax.rem(e, pos_total)
            pltpu.make_async_copy(
                x_hbm.at[pl.ds(e, _CHUNK)], xbuf.at[b], sem_x.at[b]
            ).start()
            pltpu.make_async_copy(
                pos_hbm.at[pl.ds(pe, _CHUNK)], pbuf.at[b], sem_p.at[b]
            ).start()

        def wait_in(j, b):
            e = base + j * _CHUNK
            pe = lax.rem(e, pos_total)
            pltpu.make_async_copy(
                x_hbm.at[pl.ds(e, _CHUNK)], xbuf.at[b], sem_x.at[b]
            ).wait()
            pltpu.make_async_copy(
                pos_hbm.at[pl.ds(pe, _CHUNK)], pbuf.at[b], sem_p.at[b]
            ).wait()

        def start_out(j, b):
            e = base + j * _CHUNK
            pltpu.make_async_copy(
                xbuf.at[b], out_hbm.at[pl.ds(e, _CHUNK)], sem_o.at[b]
            ).start()

        def wait_out(j, b):
            e = base + j * _CHUNK
            pltpu.make_async_copy(
                xbuf.at[b], out_hbm.at[pl.ds(e, _CHUNK)], sem_o.at[b]
            ).wait()

        start_in(0, 0)
        for j in range(n_chunks):
            b = j % 2
            wait_in(j, b)
            if j + 1 < n_chunks:
                if j >= 1:
                    wait_out(j - 1, (j + 1) % 2)
                start_in(j + 1, (j + 1) % 2)

            @plsc.parallel_loop(0, _CHUNK // 16, step=1, unroll=8)
            def add_body(i):
                s = pl.ds(i * 16, 16)
                xbuf[b, s] = xbuf[b, s] + pbuf[b, s]

            start_out(j, b)
        wait_out(n_chunks - 2, n_chunks % 2)
        wait_out(n_chunks - 1, (n_chunks - 1) % 2)

    return sc_add


@jax.jit
def kernel(x, pos_table):
    batch, seq_len, d = x.shape
    total = batch * seq_len * d
    pos_total = seq_len * d
    out = _make_sc_add(total, pos_total)(
        x.reshape(total), pos_table[:seq_len].reshape(pos_total)
    )
    return out.reshape(batch, seq_len, d)


# speedup vs baseline: 5.1585x; 5.1585x over previous
import jax
import jax.numpy as jnp
from jax.experimental import pallas as pl


def _add_block(x_ref, pos_ref, o_ref):
    o_ref[...] = x_ref[...] + pos_ref[...][None, :, :]


@jax.jit
def kernel(x, pos_table):
    batch, seq_len, d = x.shape
    s_blk = 512
    grid = (seq_len // s_blk,)
    return pl.pallas_call(
        _add_block,
        grid=grid,
        in_specs=[
            pl.BlockSpec((batch, s_blk, d), lambda s: (0, s, 0)),
            pl.BlockSpec((s_blk, d), lambda s: (s, 0)),
        ],
        out_specs=pl.BlockSpec((batch, s_blk, d), lambda s: (0, s, 0)),
        out_shape=jax.ShapeDtypeStruct((batch, seq_len, d), x.dtype),
    )(x, pos_table[:seq_len])
